# baseline (speedup 1.0000x reference)
import functools

import jax
import jax.numpy as jnp
from jax import lax
from jax.experimental import pallas as pl
from jax.experimental.pallas import tpu as pltpu
from jax.experimental.pallas import tpu_sc as plsc

B = 4096
W, C = 40, 30
MLW, MLC = 50, 50
NW = 32
LANES = 128
NWT = W // 8

_mesh = plsc.VectorSubcoreMesh(core_axis_name="c", subcore_axis_name="s")


@functools.partial(
    pl.kernel,
    mesh=_mesh,
    out_type=jax.ShapeDtypeStruct((MLW, MLC, B), jnp.float32),
    scratch_types=[
        pltpu.VMEM((257,), jnp.float32),
        pltpu.VMEM((16, 8, LANES), jnp.int32),
        pltpu.VMEM((16, 8, LANES), jnp.int32),
        pltpu.VMEM((8, 16, LANES), jnp.float32),
        pltpu.VMEM((8, 16, LANES), jnp.float32),
        pltpu.SemaphoreType.DMA,
        pltpu.SemaphoreType.DMA,
    ],
    compiler_params=pltpu.CompilerParams(needs_layout_passes=False),
)
def _sc_lookup_pad(codes_hbm, table_hbm, out_hbm, table_v, in0, in1,
                   out0, out1, insem, outsem):
    wid = lax.axis_index("s") * 2 + lax.axis_index("c")
    b0 = wid * LANES
    ins = (in0, in1)
    outs = (out0, out1)

    pltpu.sync_copy(table_hbm, table_v)

    NCROWS = (16, 14)

    def start_in(wt, cc):
        nc = NCROWS[cc]
        return pltpu.async_copy(
            codes_hbm.at[pl.ds(cc * 16, nc), pl.ds(wt * 8, 8),
                         pl.ds(b0, LANES)],
            ins[cc].at[pl.ds(0, nc), :, :],
            insem)

    def compute_slab(wt, cc):
        src, dst = ins[cc], outs[cc]

        def c_body(c, carry):
            loaded = [src[c, 0, pl.ds(l * 16, 16)] for l in range(8)]
            prev = None
            for w in range(8):
                cur = []
                nxt = []
                for l in range(8):
                    cur.append(plsc.load_gather(table_v, [loaded[l]]))
                    if w + 1 < 8:
                        nxt.append(src[c, w + 1, pl.ds(l * 16, 16)])
                    if prev is not None:
                        dst[w - 1, c, pl.ds(l * 16, 16)] = prev[l]
                prev = cur
                if nxt:
                    loaded = nxt
            for l in range(8):
                dst[7, c, pl.ds(l * 16, 16)] = prev[l]
            return carry

        lax.fori_loop(0, NCROWS[cc], c_body, 0)

    def start_out(wt, cc):
        return pltpu.async_copy(
            outs[cc],
            out_hbm.at[pl.ds(wt * 8, 8), pl.ds(cc * 16, 16),
                       pl.ds(b0, LANES)],
            outsem)

    pending_in = {0: start_in(0, 0), 1: None}
    pending_out = {0: None, 1: None}
    nxt_sched = [(0, 1)] + [(wt, cc) for wt in range(1, NWT) for cc in (0, 1)]
    k = 0
    for wt in range(NWT):
        for cc in (0, 1):
            pending_in[cc].wait()
            if k + 1 < 2 * NWT:
                nwt, ncc = nxt_sched[k]
                pending_in[ncc] = start_in(nwt, ncc)
            if pending_out[cc] is not None:
                pending_out[cc].wait()
            compute_slab(wt, cc)
            pending_out[cc] = start_out(wt, cc)
            k += 1
    pending_out[0].wait()
    pending_out[1].wait()


def kernel(char_codes, char_table):
    codes_t = jnp.transpose(char_codes, (2, 1, 0))
    out_t = _sc_lookup_pad(codes_t, char_table)
    out_t = lax.dynamic_update_slice(
        out_t, jnp.zeros((MLW, MLC - C, B), jnp.float32), (0, C, 0))
    out_t = lax.dynamic_update_slice(
        out_t, jnp.zeros((MLW - W, C, B), jnp.float32), (W, 0, 0))
    return jnp.transpose(out_t, (2, 0, 1))


# speedup vs baseline: 3.1138x; 3.1138x over previous
import functools

import jax
import jax.numpy as jnp
from jax import lax
from jax.experimental import pallas as pl
from jax.experimental.pallas import tpu as pltpu
from jax.experimental.pallas import tpu_sc as plsc

B = 4096
W, C = 40, 30
MLW, MLC = 50, 50
NW = 32
LANES = 128
NWT = W // 8

_mesh = plsc.VectorSubcoreMesh(core_axis_name="c", subcore_axis_name="s")


@functools.partial(
    pl.kernel,
    mesh=_mesh,
    out_type=jax.ShapeDtypeStruct((MLW, MLC, B), jnp.float32),
    scratch_types=[
        pltpu.VMEM((257,), jnp.float32),
        pltpu.VMEM((16, 8, LANES), jnp.int32),
        pltpu.VMEM((16, 8, LANES), jnp.int32),
        pltpu.VMEM((8, 16, LANES), jnp.float32),
        pltpu.VMEM((8, 16, LANES), jnp.float32),
        pltpu.VMEM((8, 16, LANES), jnp.float32),
        pltpu.SemaphoreType.DMA,
        pltpu.SemaphoreType.DMA,
        pltpu.SemaphoreType.DMA,
    ],
    compiler_params=pltpu.CompilerParams(needs_layout_passes=False),
)
def _sc_lookup_pad(codes_hbm, table_hbm, out_hbm, table_v, in0, in1,
                   out0, out1, zero_v, insem, outsem, zsem):
    wid = lax.axis_index("s") * 2 + lax.axis_index("c")
    b0 = wid * LANES
    ins = (in0, in1)
    outs = (out0, out1)

    pltpu.sync_copy(table_hbm, table_v)

    zv = jnp.zeros((16,), jnp.float32)

    def zfill(i, carry):
        zero_v[i // 128, (i // 8) % 16, pl.ds((i % 8) * 16, 16)] = zv
        return carry

    lax.fori_loop(0, 8 * 16 * 8, zfill, 0)
    for r in (14, 15):
        def z2(i, carry, r=r):
            out1[i // 8, r, pl.ds((i % 8) * 16, 16)] = zv
            return carry
        lax.fori_loop(0, 64, z2, 0)

    zcopies = []
    for wt in range(NWT):
        zcopies.append(pltpu.async_copy(
            zero_v,
            out_hbm.at[pl.ds(wt * 8, 8), pl.ds(32, 16), pl.ds(b0, LANES)],
            zsem))
    for ct in range(3):
        zcopies.append(pltpu.async_copy(
            zero_v,
            out_hbm.at[pl.ds(40, 8), pl.ds(ct * 16, 16), pl.ds(b0, LANES)],
            zsem))
    for ct in range(3):
        zcopies.append(pltpu.async_copy(
            zero_v.at[pl.ds(0, 2), :, :],
            out_hbm.at[pl.ds(48, 2), pl.ds(ct * 16, 16), pl.ds(b0, LANES)],
            zsem))

    NCROWS = (16, 14)

    def start_in(wt, cc):
        nc = NCROWS[cc]
        return pltpu.async_copy(
            codes_hbm.at[pl.ds(cc * 16, nc), pl.ds(wt * 8, 8),
                         pl.ds(b0, LANES)],
            ins[cc].at[pl.ds(0, nc), :, :],
            insem)

    def compute_slab(wt, cc):
        src, dst = ins[cc], outs[cc]

        def c_body(c, carry):
            loaded = [src[c, 0, pl.ds(l * 16, 16)] for l in range(8)]
            prev = None
            for w in range(8):
                cur = []
                nxt = []
                for l in range(8):
                    cur.append(plsc.load_gather(table_v, [loaded[l]]))
                    if w + 1 < 8:
                        nxt.append(src[c, w + 1, pl.ds(l * 16, 16)])
                    if prev is not None:
                        dst[w - 1, c, pl.ds(l * 16, 16)] = prev[l]
                prev = cur
                if nxt:
                    loaded = nxt
            for l in range(8):
                dst[7, c, pl.ds(l * 16, 16)] = prev[l]
            return carry

        lax.fori_loop(0, NCROWS[cc], c_body, 0)

    def start_out(wt, cc):
        return pltpu.async_copy(
            outs[cc],
            out_hbm.at[pl.ds(wt * 8, 8), pl.ds(cc * 16, 16),
                       pl.ds(b0, LANES)],
            outsem)

    pending_in = {0: start_in(0, 0), 1: None}
    pending_out = {0: None, 1: None}
    nxt_sched = [(0, 1)] + [(wt, cc) for wt in range(1, NWT) for cc in (0, 1)]
    k = 0
    for wt in range(NWT):
        for cc in (0, 1):
            pending_in[cc].wait()
            if k + 1 < 2 * NWT:
                nwt, ncc = nxt_sched[k]
                pending_in[ncc] = start_in(nwt, ncc)
            if pending_out[cc] is not None:
                pending_out[cc].wait()
            compute_slab(wt, cc)
            pending_out[cc] = start_out(wt, cc)
            k += 1
    pending_out[0].wait()
    pending_out[1].wait()
    for zc in zcopies:
        zc.wait()


def kernel(char_codes, char_table):
    codes_t = jnp.transpose(char_codes, (2, 1, 0))
    out_t = _sc_lookup_pad(codes_t, char_table)
    out_t = lax.dynamic_update_slice(
        out_t, jnp.zeros((MLW, 2, B), jnp.float32), (0, MLC - 2, 0))
    return jnp.transpose(out_t, (2, 0, 1))


# speedup vs baseline: 3.1789x; 1.0209x over previous
import functools

import jax
import jax.numpy as jnp
from jax import lax
from jax.experimental import pallas as pl
from jax.experimental.pallas import tpu as pltpu
from jax.experimental.pallas import tpu_sc as plsc

B = 4096
W, C = 40, 30
MLW, MLC = 50, 50
NW = 32
LANES = 128
NWT = W // 8

_mesh = plsc.VectorSubcoreMesh(core_axis_name="c", subcore_axis_name="s")


@functools.partial(
    pl.kernel,
    mesh=_mesh,
    out_type=jax.ShapeDtypeStruct((MLW, MLC, B), jnp.float32),
    scratch_types=[
        pltpu.VMEM((257,), jnp.float32),
        pltpu.VMEM((16, 8, LANES), jnp.int32),
        pltpu.VMEM((16, 8, LANES), jnp.int32),
        pltpu.VMEM((8, 16, LANES), jnp.float32),
        pltpu.VMEM((8, 16, LANES), jnp.float32),
        pltpu.VMEM((8, 16, LANES), jnp.float32),
        pltpu.SemaphoreType.DMA,
        pltpu.SemaphoreType.DMA,
        pltpu.SemaphoreType.DMA,
    ],
    compiler_params=pltpu.CompilerParams(needs_layout_passes=False),
)
def _sc_lookup_pad(codes_hbm, table_hbm, out_hbm, table_v, in0, in1,
                   out0, out1, zero_v, insem, outsem, zsem):
    wid = lax.axis_index("s") * 2 + lax.axis_index("c")
    b0 = wid * LANES
    ins = (in0, in1)
    outs = (out0, out1)

    pltpu.sync_copy(table_hbm, table_v)

    zv = jnp.zeros((16,), jnp.float32)

    def zfill(i, carry):
        zero_v[i // 128, (i // 8) % 16, pl.ds((i % 8) * 16, 16)] = zv
        return carry

    lax.fori_loop(0, 8 * 16 * 8, zfill, 0)
    for r in (14, 15):
        def z2(i, carry, r=r):
            out1[i // 8, r, pl.ds((i % 8) * 16, 16)] = zv
            return carry
        lax.fori_loop(0, 64, z2, 0)

    def make_zero_starts():
        starts = []
        for wt in range(NWT):
            starts.append(lambda wt=wt: pltpu.async_copy(
                zero_v,
                out_hbm.at[pl.ds(wt * 8, 8), pl.ds(32, 16), pl.ds(b0, LANES)],
                zsem))
        for ct in range(3):
            starts.append(lambda ct=ct: pltpu.async_copy(
                zero_v,
                out_hbm.at[pl.ds(40, 8), pl.ds(ct * 16, 16),
                           pl.ds(b0, LANES)],
                zsem))
        for ct in range(3):
            starts.append(lambda ct=ct: pltpu.async_copy(
                zero_v.at[pl.ds(0, 2), :, :],
                out_hbm.at[pl.ds(48, 2), pl.ds(ct * 16, 16),
                           pl.ds(b0, LANES)],
                zsem))
        return starts

    zero_starts = make_zero_starts()
    zcopies = []

    NCROWS = (16, 14)

    def start_in(wt, cc):
        nc = NCROWS[cc]
        return pltpu.async_copy(
            codes_hbm.at[pl.ds(cc * 16, nc), pl.ds(wt * 8, 8),
                         pl.ds(b0, LANES)],
            ins[cc].at[pl.ds(0, nc), :, :],
            insem)

    def compute_slab(wt, cc):
        src, dst = ins[cc], outs[cc]

        def c_body(c, carry):
            loaded = [src[c, 0, pl.ds(l * 16, 16)] for l in range(8)]
            prev = None
            for w in range(8):
                cur = []
                nxt = []
                for l in range(8):
                    cur.append(plsc.load_gather(table_v, [loaded[l]]))
                    if w + 1 < 8:
                        nxt.append(src[c, w + 1, pl.ds(l * 16, 16)])
                    if prev is not None:
                        dst[w - 1, c, pl.ds(l * 16, 16)] = prev[l]
                prev = cur
                if nxt:
                    loaded = nxt
            for l in range(8):
                dst[7, c, pl.ds(l * 16, 16)] = prev[l]
            return carry

        lax.fori_loop(0, NCROWS[cc], c_body, 0)

    def start_out(wt, cc):
        return pltpu.async_copy(
            outs[cc],
            out_hbm.at[pl.ds(wt * 8, 8), pl.ds(cc * 16, 16),
                       pl.ds(b0, LANES)],
            outsem)

    pending_in = {0: start_in(0, 0), 1: None}
    pending_out = {0: None, 1: None}
    nxt_sched = [(0, 1)] + [(wt, cc) for wt in range(1, NWT) for cc in (0, 1)]
    k = 0
    for wt in range(NWT):
        for cc in (0, 1):
            pending_in[cc].wait()
            if k + 1 < 2 * NWT:
                nwt, ncc = nxt_sched[k]
                pending_in[ncc] = start_in(nwt, ncc)
            if pending_out[cc] is not None:
                pending_out[cc].wait()
            compute_slab(wt, cc)
            pending_out[cc] = start_out(wt, cc)
            if k < len(zero_starts):
                zcopies.append(zero_starts[k]())
            k += 1
    for zs in zero_starts[k:]:
        zcopies.append(zs())
    pending_out[0].wait()
    pending_out[1].wait()
    for zc in zcopies:
        zc.wait()


def kernel(char_codes, char_table):
    codes_t = jnp.transpose(char_codes, (2, 1, 0))
    out_t = _sc_lookup_pad(codes_t, char_table)
    out_t = lax.dynamic_update_slice(
        out_t, jnp.zeros((MLW, 2, B), jnp.float32), (0, MLC - 2, 0))
    return jnp.transpose(out_t, (2, 0, 1))
